# baseline (speedup 1.0000x reference)
import functools

import numpy as np
import jax
import jax.numpy as jnp
from jax import lax
from jax.experimental import pallas as pl
from jax.experimental.pallas import tpu as pltpu
from jax.experimental.pallas import tpu_sc as plsc

N_NODES = 10000
POS_CH = 16
EMB = 2 * POS_CH * 3
KIN = 2 * EMB
HID = 128
NUM_LAYERS = 4
SC_WORKERS = 32
SC_CHUNK = 128
BE = 256
BN = 1000


def _round_up(v, m):
    return (v + m - 1) // m * m


def _gelu(v):
    return 0.5 * v * (1.0 + lax.erf(v * np.float32(1.0 / np.sqrt(2.0))))


def _sc_gather(table, idx):
    b_total = idx.shape[0]
    d = table.shape[1]
    b_per_w = b_total // SC_WORKERS
    chunks = b_per_w // SC_CHUNK
    mesh = plsc.VectorSubcoreMesh(core_axis_name="c", subcore_axis_name="s")

    nbuf = min(4, chunks)

    @functools.partial(
        pl.kernel,
        out_type=jax.ShapeDtypeStruct((b_total, d), table.dtype),
        mesh=mesh,
        scratch_types=(
            [pltpu.VMEM((b_per_w,), jnp.int32)]
            + [pltpu.VMEM((SC_CHUNK, d), table.dtype)] * nbuf
            + [pltpu.SemaphoreType.DMA] * (2 * nbuf)
        ),
    )
    def gather_kernel(table_hbm, idx_hbm, out_hbm, idx_v, *rest):
        bufs = rest[:nbuf]
        gsem = rest[nbuf:2 * nbuf]
        wsem = rest[2 * nbuf:]
        wid = lax.axis_index("s") * 2 + lax.axis_index("c")
        base = wid * b_per_w
        pltpu.sync_copy(idx_hbm.at[pl.ds(base, b_per_w)], idx_v)

        def gather_chunk(j):
            return pltpu.async_copy(
                table_hbm.at[idx_v.at[pl.ds(j * SC_CHUNK, SC_CHUNK)]],
                bufs[j % nbuf], gsem[j % nbuf])

        cps = [None] * nbuf
        wbs = [None] * nbuf
        for j in range(nbuf - 1):
            cps[j % nbuf] = gather_chunk(j)
        for j in range(chunks):
            cps[j % nbuf].wait()
            wbs[j % nbuf] = pltpu.async_copy(
                bufs[j % nbuf],
                out_hbm.at[pl.ds(base + j * SC_CHUNK, SC_CHUNK)],
                wsem[j % nbuf])
            nj = j + nbuf - 1
            if nj < chunks:
                if wbs[nj % nbuf] is not None:
                    wbs[nj % nbuf].wait()
                    wbs[nj % nbuf] = None
                cps[nj % nbuf] = gather_chunk(nj)
        for wb in wbs:
            if wb is not None:
                wb.wait()

    return gather_kernel(table, idx)


def _mlp2_body(x_ref, w1_ref, b1_ref, w2_ref, b2_ref, o_ref):
    t = _gelu(jnp.dot(x_ref[...], w1_ref[...],
                      preferred_element_type=jnp.float32) + b1_ref[...])
    o_ref[...] = jnp.dot(t, w2_ref[...],
                         preferred_element_type=jnp.float32) + b2_ref[...]


def _enc_body(x_ref, w1_ref, b1_ref, w2_ref, b2_ref, fr_ref, ph_ref,
              h_ref, pe_ref):
    x = x_ref[...]
    t = _gelu(jnp.dot(x, w1_ref[...],
                      preferred_element_type=jnp.float32) + b1_ref[...])
    h_ref[...] = jnp.dot(t, w2_ref[...],
                         preferred_element_type=jnp.float32) + b2_ref[...]
    cols = [jnp.broadcast_to(x[:, c:c + 1], (x.shape[0], 32))
            for c in range(3)]
    pos_big = jnp.concatenate(cols, axis=1)
    g = jnp.sin(pos_big * fr_ref[...] + ph_ref[...])
    pe_ref[...] = jnp.pad(g, ((0, 0), (0, 32)))


NR = BE + 8


def _edge_body(n_edges, with_cnt, n0s_ref, ped_ref, pes_ref, hd_ref, src_ref,
               w0d_ref, w0s_ref, b0_ref, w1_ref, b1_ref, w2_ref, b2_ref,
               acc_ref, *cnt_out):
    i = pl.program_id(0)

    @pl.when(i == 0)
    def _():
        acc_ref[...] = jnp.zeros_like(acc_ref)
        if with_cnt:
            cnt_out[0][...] = jnp.zeros_like(cnt_out[0])

    t = (jnp.dot(ped_ref[...].astype(jnp.bfloat16), w0d_ref[...],
                 preferred_element_type=jnp.float32)
         + jnp.dot(pes_ref[...].astype(jnp.bfloat16), w0s_ref[...],
                   preferred_element_type=jnp.float32)
         + b0_ref[...])
    t = _gelu(t).astype(jnp.bfloat16)
    t = _gelu(jnp.dot(t, w1_ref[...],
                      preferred_element_type=jnp.float32) + b1_ref[...])
    k = jnp.dot(t.astype(jnp.bfloat16), w2_ref[...],
                preferred_element_type=jnp.float32) + b2_ref[...]

    msg = k * hd_ref[...]
    rows = i * BE + lax.broadcasted_iota(jnp.int32, (BE, 1), 0)
    msg = jnp.where(rows < n_edges, msg, 0.0)

    n0 = n0s_ref[i]
    src_rel = src_ref[0, 0, :].reshape(1, BE) - n0
    oh = (lax.broadcasted_iota(jnp.int32, (NR, BE), 0)
          == src_rel).astype(jnp.bfloat16)
    upd = jnp.dot(oh, msg.astype(jnp.bfloat16),
                  preferred_element_type=jnp.float32)
    acc_ref[pl.ds(n0, NR), :] += upd
    if with_cnt:
        onesm = jnp.where(rows < n_edges,
                          jnp.float32(1.0), jnp.float32(0.0))
        cnt_out[0][pl.ds(n0, NR), :] += jnp.dot(
            oh, jnp.broadcast_to(onesm, (BE, HID)).astype(jnp.bfloat16),
            preferred_element_type=jnp.float32)


def _update_body(h_ref, seg_ref, cnt_ref, gam_ref, bet_ref, o_ref):
    inv = 1.0 / jnp.maximum(cnt_ref[:, 0:1], 1.0)
    hn = h_ref[...] + seg_ref[...] * inv
    mu = jnp.mean(hn, axis=1, keepdims=True)
    var = jnp.mean((hn - mu) ** 2, axis=1, keepdims=True)
    o_ref[...] = ((hn - mu) * lax.rsqrt(var + 1e-5) * gam_ref[...]
                  + bet_ref[...])


def _full(shape):
    return pl.BlockSpec(shape, lambda i: (0, 0))


def kernel(x, params, edge_src, edge_dst):
    n_edges = edge_src.shape[0]
    ep = _round_up(n_edges, SC_WORKERS * SC_CHUNK)
    spread = (jnp.arange(ep - n_edges, dtype=jnp.int32) * 7919) % N_NODES
    dstp = jnp.concatenate([edge_dst.astype(jnp.int32), spread])
    srcp = jnp.concatenate([edge_src.astype(jnp.int32), spread])

    n0s = (srcp[::BE] // 8) * 8
    srcb = srcp.reshape(-1, 1, BE)

    freqs = 1.0 / (10000.0 ** (np.arange(POS_CH, dtype=np.float32) / POS_CH))
    fr_half = np.concatenate([freqs, freqs])
    ph_half = np.concatenate([np.zeros(POS_CH, np.float32),
                              np.full(POS_CH, np.pi / 2, np.float32)])
    fr96 = jnp.asarray(np.tile(fr_half, 3)[None, :])
    ph96 = jnp.asarray(np.tile(ph_half, 3)[None, :])

    def row(v):
        return v.reshape(1, -1)

    h, pe = pl.pallas_call(
        _enc_body,
        grid=(N_NODES // BN,),
        in_specs=[
            pl.BlockSpec((BN, HID), lambda i: (i, 0)),
            _full((HID, HID)), _full((1, HID)),
            _full((HID, HID)), _full((1, HID)),
            _full((1, EMB)), _full((1, EMB)),
        ],
        out_specs=[pl.BlockSpec((BN, HID), lambda i: (i, 0)),
                   pl.BlockSpec((BN, HID), lambda i: (i, 0))],
        out_shape=[jax.ShapeDtypeStruct((N_NODES, HID), jnp.float32),
                   jax.ShapeDtypeStruct((N_NODES, HID), jnp.float32)],
    )(x, params['enc_w1'], row(params['enc_b1']),
      params['enc_w2'], row(params['enc_b2']), fr96, ph96)

    peg = _sc_gather(pe, jnp.concatenate([dstp, srcp]))

    ne_blocks = ep // BE
    n_pad = _round_up(N_NODES + NR, 8)
    cnt = None
    for l in range(NUM_LAYERS):
        hd = _sc_gather(h, dstp)

        w0 = params[f'k{l}_w0']
        w0d = jnp.pad(w0[:EMB], ((0, HID - EMB), (0, 0))).astype(jnp.bfloat16)
        w0s = jnp.pad(w0[EMB:], ((0, HID - EMB), (0, 0))).astype(jnp.bfloat16)
        w1b = params[f'k{l}_w1'].astype(jnp.bfloat16)
        w2b = params[f'k{l}_w2'].astype(jnp.bfloat16)

        with_cnt = l == 0
        out_specs = [pl.BlockSpec((n_pad, HID), lambda i: (0, 0))]
        out_shape = [jax.ShapeDtypeStruct((n_pad, HID), jnp.float32)]
        if with_cnt:
            out_specs.append(pl.BlockSpec((n_pad, HID), lambda i: (0, 0)))
            out_shape.append(jax.ShapeDtypeStruct((n_pad, HID), jnp.float32))
        res = pl.pallas_call(
            functools.partial(_edge_body, n_edges, with_cnt),
            grid=(ne_blocks,),
            in_specs=[
                pl.BlockSpec(memory_space=pltpu.SMEM),
                pl.BlockSpec((BE, HID), lambda i: (i, 0)),
                pl.BlockSpec((BE, HID), lambda i: (i + ne_blocks, 0)),
                pl.BlockSpec((BE, HID), lambda i: (i, 0)),
                pl.BlockSpec((1, 1, BE), lambda i: (i, 0, 0)),
                _full((HID, HID)), _full((HID, HID)), _full((1, HID)),
                _full((HID, 2 * HID)), _full((1, 2 * HID)),
                _full((2 * HID, HID)), _full((1, HID)),
            ],
            out_specs=out_specs,
            out_shape=out_shape,
        )(n0s, peg, peg, hd, srcb,
          w0d, w0s, row(params[f'k{l}_b0']),
          w1b, row(params[f'k{l}_b1']),
          w2b, row(params[f'k{l}_b2']))
        if with_cnt:
            seg, cnt = res
        else:
            seg, = res

        h = pl.pallas_call(
            _update_body,
            grid=(N_NODES // BN,),
            in_specs=[
                pl.BlockSpec((BN, HID), lambda i: (i, 0)),
                pl.BlockSpec((BN, HID), lambda i: (i, 0)),
                pl.BlockSpec((BN, HID), lambda i: (i, 0)),
                _full((1, HID)), _full((1, HID)),
            ],
            out_specs=pl.BlockSpec((BN, HID), lambda i: (i, 0)),
            out_shape=jax.ShapeDtypeStruct((N_NODES, HID), jnp.float32),
        )(h, seg, cnt, row(params[f'ln{l}_g']), row(params[f'ln{l}_b']))

    hw2 = jnp.pad(params['head_w2'], ((0, 0), (0, 5)))
    hb2 = jnp.pad(params['head_b2'], (0, 5))
    out = pl.pallas_call(
        _mlp2_body,
        grid=(N_NODES // BN,),
        in_specs=[
            pl.BlockSpec((BN, HID), lambda i: (i, 0)),
            _full((HID, HID)), _full((1, HID)),
            _full((HID, 8)), _full((1, 8)),
        ],
        out_specs=pl.BlockSpec((BN, 8), lambda i: (i, 0)),
        out_shape=jax.ShapeDtypeStruct((N_NODES, 8), jnp.float32),
    )(h, params['head_w1'], row(params['head_b1']), hw2, row(hb2))

    return out[:, :3]


# speedup vs baseline: 1.1229x; 1.1229x over previous
import functools

import numpy as np
import jax
import jax.numpy as jnp
from jax import lax
from jax.experimental import pallas as pl
from jax.experimental.pallas import tpu as pltpu
from jax.experimental.pallas import tpu_sc as plsc

N_NODES = 10000
POS_CH = 16
EMB = 2 * POS_CH * 3
KIN = 2 * EMB
HID = 128
NUM_LAYERS = 4
SC_WORKERS = 32
SC_CHUNK = 128
BE = 512
BN = 1000


def _round_up(v, m):
    return (v + m - 1) // m * m


def _gelu(v):
    return 0.5 * v * (1.0 + lax.erf(v * np.float32(1.0 / np.sqrt(2.0))))


def _sc_gather(table, idx):
    b_total = idx.shape[0]
    d = table.shape[1]
    b_per_w = b_total // SC_WORKERS
    chunks = b_per_w // SC_CHUNK
    mesh = plsc.VectorSubcoreMesh(core_axis_name="c", subcore_axis_name="s")

    nbuf = min(4, chunks)

    @functools.partial(
        pl.kernel,
        out_type=jax.ShapeDtypeStruct((b_total, d), table.dtype),
        mesh=mesh,
        scratch_types=(
            [pltpu.VMEM((b_per_w,), jnp.int32)]
            + [pltpu.VMEM((SC_CHUNK, d), table.dtype)] * nbuf
            + [pltpu.SemaphoreType.DMA] * (2 * nbuf)
        ),
    )
    def gather_kernel(table_hbm, idx_hbm, out_hbm, idx_v, *rest):
        bufs = rest[:nbuf]
        gsem = rest[nbuf:2 * nbuf]
        wsem = rest[2 * nbuf:]
        wid = lax.axis_index("s") * 2 + lax.axis_index("c")
        base = wid * b_per_w
        pltpu.sync_copy(idx_hbm.at[pl.ds(base, b_per_w)], idx_v)

        def gather_chunk(j):
            return pltpu.async_copy(
                table_hbm.at[idx_v.at[pl.ds(j * SC_CHUNK, SC_CHUNK)]],
                bufs[j % nbuf], gsem[j % nbuf])

        cps = [None] * nbuf
        wbs = [None] * nbuf
        for j in range(nbuf - 1):
            cps[j % nbuf] = gather_chunk(j)
        for j in range(chunks):
            cps[j % nbuf].wait()
            wbs[j % nbuf] = pltpu.async_copy(
                bufs[j % nbuf],
                out_hbm.at[pl.ds(base + j * SC_CHUNK, SC_CHUNK)],
                wsem[j % nbuf])
            nj = j + nbuf - 1
            if nj < chunks:
                if wbs[nj % nbuf] is not None:
                    wbs[nj % nbuf].wait()
                    wbs[nj % nbuf] = None
                cps[nj % nbuf] = gather_chunk(nj)
        for wb in wbs:
            if wb is not None:
                wb.wait()

    return gather_kernel(table, idx)


def _mlp2_body(x_ref, w1_ref, b1_ref, w2_ref, b2_ref, o_ref):
    t = _gelu(jnp.dot(x_ref[...], w1_ref[...],
                      preferred_element_type=jnp.float32) + b1_ref[...])
    o_ref[...] = jnp.dot(t, w2_ref[...],
                         preferred_element_type=jnp.float32) + b2_ref[...]


def _enc_body(x_ref, w1_ref, b1_ref, w2_ref, b2_ref, fr_ref, ph_ref,
              h_ref, pe_ref):
    x = x_ref[...]
    t = _gelu(jnp.dot(x, w1_ref[...],
                      preferred_element_type=jnp.float32) + b1_ref[...])
    h_ref[...] = jnp.dot(t, w2_ref[...],
                         preferred_element_type=jnp.float32) + b2_ref[...]
    cols = [jnp.broadcast_to(x[:, c:c + 1], (x.shape[0], 32))
            for c in range(3)]
    pos_big = jnp.concatenate(cols, axis=1)
    g = jnp.sin(pos_big * fr_ref[...] + ph_ref[...])
    pe_ref[...] = jnp.pad(g, ((0, 0), (0, 32)))


NR = BE + 8


def _kmlp_body(ped_ref, pes_ref,
               w0d_ref, w0s_ref, b0_ref, w1_ref, b1_ref, w2_ref, b2_ref,
               k_ref):
    t = (jnp.dot(ped_ref[...].astype(jnp.bfloat16), w0d_ref[...],
                 preferred_element_type=jnp.float32)
         + jnp.dot(pes_ref[...].astype(jnp.bfloat16), w0s_ref[...],
                   preferred_element_type=jnp.float32)
         + b0_ref[...])
    t = _gelu(t).astype(jnp.bfloat16)
    t = _gelu(jnp.dot(t, w1_ref[...],
                      preferred_element_type=jnp.float32) + b1_ref[...])
    k = jnp.dot(t.astype(jnp.bfloat16), w2_ref[...],
                preferred_element_type=jnp.float32) + b2_ref[...]
    k_ref[...] = k.astype(jnp.bfloat16)


def _scatter_body(n_edges, with_cnt, n0s_ref, k_ref, hd_ref, src_ref,
                  acc_ref, *cnt_out):
    i = pl.program_id(0)

    @pl.when(i == 0)
    def _():
        acc_ref[...] = jnp.zeros_like(acc_ref)
        if with_cnt:
            cnt_out[0][...] = jnp.zeros_like(cnt_out[0])

    msg = k_ref[...].astype(jnp.float32) * hd_ref[...]
    rows = i * BE + lax.broadcasted_iota(jnp.int32, (BE, 1), 0)
    msg = jnp.where(rows < n_edges, msg, 0.0)

    n0 = n0s_ref[i]
    src_rel = src_ref[0, 0, :].reshape(1, BE) - n0
    oh = (lax.broadcasted_iota(jnp.int32, (NR, BE), 0)
          == src_rel).astype(jnp.bfloat16)
    upd = jnp.dot(oh, msg.astype(jnp.bfloat16),
                  preferred_element_type=jnp.float32)
    acc_ref[pl.ds(n0, NR), :] += upd
    if with_cnt:
        onesm = jnp.where(rows < n_edges,
                          jnp.float32(1.0), jnp.float32(0.0))
        cnt_out[0][pl.ds(n0, NR), :] += jnp.dot(
            oh, jnp.broadcast_to(onesm, (BE, HID)).astype(jnp.bfloat16),
            preferred_element_type=jnp.float32)


def _update_body(h_ref, seg_ref, cnt_ref, gam_ref, bet_ref, o_ref):
    inv = 1.0 / jnp.maximum(cnt_ref[:, 0:1], 1.0)
    hn = h_ref[...] + seg_ref[...] * inv
    mu = jnp.mean(hn, axis=1, keepdims=True)
    var = jnp.mean((hn - mu) ** 2, axis=1, keepdims=True)
    o_ref[...] = ((hn - mu) * lax.rsqrt(var + 1e-5) * gam_ref[...]
                  + bet_ref[...])


def _full(shape):
    return pl.BlockSpec(shape, lambda i: (0, 0))


def kernel(x, params, edge_src, edge_dst):
    n_edges = edge_src.shape[0]
    ep = _round_up(n_edges, SC_WORKERS * SC_CHUNK)
    spread = (jnp.arange(ep - n_edges, dtype=jnp.int32) * 7919) % N_NODES
    dstp = jnp.concatenate([edge_dst.astype(jnp.int32), spread])
    srcp = jnp.concatenate([edge_src.astype(jnp.int32), spread])

    n0s = (srcp[::BE] // 8) * 8
    srcb = srcp.reshape(-1, 1, BE)

    freqs = 1.0 / (10000.0 ** (np.arange(POS_CH, dtype=np.float32) / POS_CH))
    fr_half = np.concatenate([freqs, freqs])
    ph_half = np.concatenate([np.zeros(POS_CH, np.float32),
                              np.full(POS_CH, np.pi / 2, np.float32)])
    fr96 = jnp.asarray(np.tile(fr_half, 3)[None, :])
    ph96 = jnp.asarray(np.tile(ph_half, 3)[None, :])

    def row(v):
        return v.reshape(1, -1)

    h, pe = pl.pallas_call(
        _enc_body,
        grid=(N_NODES // BN,),
        in_specs=[
            pl.BlockSpec((BN, HID), lambda i: (i, 0)),
            _full((HID, HID)), _full((1, HID)),
            _full((HID, HID)), _full((1, HID)),
            _full((1, EMB)), _full((1, EMB)),
        ],
        out_specs=[pl.BlockSpec((BN, HID), lambda i: (i, 0)),
                   pl.BlockSpec((BN, HID), lambda i: (i, 0))],
        out_shape=[jax.ShapeDtypeStruct((N_NODES, HID), jnp.float32),
                   jax.ShapeDtypeStruct((N_NODES, HID), jnp.float32)],
    )(x, params['enc_w1'], row(params['enc_b1']),
      params['enc_w2'], row(params['enc_b2']), fr96, ph96)

    peg = _sc_gather(pe, jnp.concatenate([dstp, srcp]))

    ne_blocks = ep // BE
    n_pad = _round_up(N_NODES + NR, 8)
    cnt = None
    for l in range(NUM_LAYERS):
        hd = _sc_gather(h, dstp)

        w0 = params[f'k{l}_w0']
        w0d = jnp.pad(w0[:EMB], ((0, HID - EMB), (0, 0))).astype(jnp.bfloat16)
        w0s = jnp.pad(w0[EMB:], ((0, HID - EMB), (0, 0))).astype(jnp.bfloat16)
        w1b = params[f'k{l}_w1'].astype(jnp.bfloat16)
        w2b = params[f'k{l}_w2'].astype(jnp.bfloat16)

        kk = pl.pallas_call(
            _kmlp_body,
            grid=(ne_blocks,),
            in_specs=[
                pl.BlockSpec((BE, HID), lambda i: (i, 0)),
                pl.BlockSpec((BE, HID), lambda i: (i + ne_blocks, 0)),
                _full((HID, HID)), _full((HID, HID)), _full((1, HID)),
                _full((HID, 2 * HID)), _full((1, 2 * HID)),
                _full((2 * HID, HID)), _full((1, HID)),
            ],
            out_specs=pl.BlockSpec((BE, HID), lambda i: (i, 0)),
            out_shape=jax.ShapeDtypeStruct((ep, HID), jnp.bfloat16),
        )(peg, peg,
          w0d, w0s, row(params[f'k{l}_b0']),
          w1b, row(params[f'k{l}_b1']),
          w2b, row(params[f'k{l}_b2']))

        with_cnt = l == 0
        out_specs = [pl.BlockSpec((n_pad, HID), lambda i: (0, 0))]
        out_shape = [jax.ShapeDtypeStruct((n_pad, HID), jnp.float32)]
        if with_cnt:
            out_specs.append(pl.BlockSpec((n_pad, HID), lambda i: (0, 0)))
            out_shape.append(jax.ShapeDtypeStruct((n_pad, HID), jnp.float32))
        res = pl.pallas_call(
            functools.partial(_scatter_body, n_edges, with_cnt),
            grid=(ne_blocks,),
            in_specs=[
                pl.BlockSpec(memory_space=pltpu.SMEM),
                pl.BlockSpec((BE, HID), lambda i: (i, 0)),
                pl.BlockSpec((BE, HID), lambda i: (i, 0)),
                pl.BlockSpec((1, 1, BE), lambda i: (i, 0, 0)),
            ],
            out_specs=out_specs,
            out_shape=out_shape,
        )(n0s, kk, hd, srcb)
        if with_cnt:
            seg, cnt = res
        else:
            seg, = res

        h = pl.pallas_call(
            _update_body,
            grid=(N_NODES // BN,),
            in_specs=[
                pl.BlockSpec((BN, HID), lambda i: (i, 0)),
                pl.BlockSpec((BN, HID), lambda i: (i, 0)),
                pl.BlockSpec((BN, HID), lambda i: (i, 0)),
                _full((1, HID)), _full((1, HID)),
            ],
            out_specs=pl.BlockSpec((BN, HID), lambda i: (i, 0)),
            out_shape=jax.ShapeDtypeStruct((N_NODES, HID), jnp.float32),
        )(h, seg, cnt, row(params[f'ln{l}_g']), row(params[f'ln{l}_b']))

    hw2 = jnp.pad(params['head_w2'], ((0, 0), (0, 5)))
    hb2 = jnp.pad(params['head_b2'], (0, 5))
    out = pl.pallas_call(
        _mlp2_body,
        grid=(N_NODES // BN,),
        in_specs=[
            pl.BlockSpec((BN, HID), lambda i: (i, 0)),
            _full((HID, HID)), _full((1, HID)),
            _full((HID, 8)), _full((1, 8)),
        ],
        out_specs=pl.BlockSpec((BN, 8), lambda i: (i, 0)),
        out_shape=jax.ShapeDtypeStruct((N_NODES, 8), jnp.float32),
    )(h, params['head_w1'], row(params['head_b1']), hw2, row(hb2))

    return out[:, :3]


# speedup vs baseline: 1.4086x; 1.2544x over previous
import functools

import numpy as np
import jax
import jax.numpy as jnp
from jax import lax
from jax.experimental import pallas as pl
from jax.experimental.pallas import tpu as pltpu
from jax.experimental.pallas import tpu_sc as plsc

N_NODES = 10000
POS_CH = 16
EMB = 2 * POS_CH * 3
KIN = 2 * EMB
HID = 128
NUM_LAYERS = 4
SC_WORKERS = 32
SC_CHUNK = 128
BE = 512
BN = 1000


def _round_up(v, m):
    return (v + m - 1) // m * m


def _gelu(v):
    return 0.5 * v * (1.0 + lax.erf(v * np.float32(1.0 / np.sqrt(2.0))))


def _sc_gather(table, idx):
    b_total = idx.shape[0]
    d = table.shape[1]
    b_per_w = b_total // SC_WORKERS
    chunks = b_per_w // SC_CHUNK
    mesh = plsc.VectorSubcoreMesh(core_axis_name="c", subcore_axis_name="s")

    nbuf = min(4, chunks)

    @functools.partial(
        pl.kernel,
        out_type=jax.ShapeDtypeStruct((b_total, d), table.dtype),
        mesh=mesh,
        scratch_types=(
            [pltpu.VMEM((b_per_w,), jnp.int32)]
            + [pltpu.VMEM((SC_CHUNK, d), table.dtype)] * nbuf
            + [pltpu.SemaphoreType.DMA] * (2 * nbuf)
        ),
    )
    def gather_kernel(table_hbm, idx_hbm, out_hbm, idx_v, *rest):
        bufs = rest[:nbuf]
        gsem = rest[nbuf:2 * nbuf]
        wsem = rest[2 * nbuf:]
        wid = lax.axis_index("s") * 2 + lax.axis_index("c")
        base = wid * b_per_w
        pltpu.sync_copy(idx_hbm.at[pl.ds(base, b_per_w)], idx_v)

        def gather_chunk(j):
            return pltpu.async_copy(
                table_hbm.at[idx_v.at[pl.ds(j * SC_CHUNK, SC_CHUNK)]],
                bufs[j % nbuf], gsem[j % nbuf])

        cps = [None] * nbuf
        wbs = [None] * nbuf
        for j in range(nbuf - 1):
            cps[j % nbuf] = gather_chunk(j)
        for j in range(chunks):
            cps[j % nbuf].wait()
            wbs[j % nbuf] = pltpu.async_copy(
                bufs[j % nbuf],
                out_hbm.at[pl.ds(base + j * SC_CHUNK, SC_CHUNK)],
                wsem[j % nbuf])
            nj = j + nbuf - 1
            if nj < chunks:
                if wbs[nj % nbuf] is not None:
                    wbs[nj % nbuf].wait()
                    wbs[nj % nbuf] = None
                cps[nj % nbuf] = gather_chunk(nj)
        for wb in wbs:
            if wb is not None:
                wb.wait()

    return gather_kernel(table, idx)


def _mlp2_body(x_ref, w1_ref, b1_ref, w2_ref, b2_ref, o_ref):
    t = _gelu(jnp.dot(x_ref[...], w1_ref[...],
                      preferred_element_type=jnp.float32) + b1_ref[...])
    o_ref[...] = jnp.dot(t, w2_ref[...],
                         preferred_element_type=jnp.float32) + b2_ref[...]


def _enc_body(x_ref, w1_ref, b1_ref, w2_ref, b2_ref, fr_ref, ph_ref,
              h_ref, pe_ref):
    x = x_ref[...]
    t = _gelu(jnp.dot(x, w1_ref[...],
                      preferred_element_type=jnp.float32) + b1_ref[...])
    h_ref[...] = jnp.dot(t, w2_ref[...],
                         preferred_element_type=jnp.float32) + b2_ref[...]
    cols = [jnp.broadcast_to(x[:, c:c + 1], (x.shape[0], 32))
            for c in range(3)]
    pos_big = jnp.concatenate(cols, axis=1)
    g = jnp.sin(pos_big * fr_ref[...] + ph_ref[...])
    pe_ref[...] = jnp.pad(g, ((0, 0), (0, 32)))


NR = BE + 8


def _edge_body(n_edges, with_cnt, n0s_ref, ped_ref, pes_ref, hd_ref, src_ref,
               w0d_ref, w0s_ref, b0_ref, w1_ref, b1_ref, w2_ref, b2_ref,
               acc_ref, *cnt_out):
    i = pl.program_id(0)

    @pl.when(i == 0)
    def _():
        acc_ref[...] = jnp.zeros_like(acc_ref)
        if with_cnt:
            cnt_out[0][...] = jnp.zeros_like(cnt_out[0])

    t = (jnp.dot(ped_ref[...].astype(jnp.bfloat16), w0d_ref[...],
                 preferred_element_type=jnp.float32)
         + jnp.dot(pes_ref[...].astype(jnp.bfloat16), w0s_ref[...],
                   preferred_element_type=jnp.float32)
         + b0_ref[...])
    t = _gelu(t).astype(jnp.bfloat16)
    t = _gelu(jnp.dot(t, w1_ref[...],
                      preferred_element_type=jnp.float32) + b1_ref[...])
    k = jnp.dot(t.astype(jnp.bfloat16), w2_ref[...],
                preferred_element_type=jnp.float32) + b2_ref[...]

    msg = k * hd_ref[...]
    rows = i * BE + lax.broadcasted_iota(jnp.int32, (BE, 1), 0)
    msg = jnp.where(rows < n_edges, msg, 0.0)

    n0 = n0s_ref[i]
    src_rel = src_ref[0, 0, :].reshape(1, BE) - n0
    oh = (lax.broadcasted_iota(jnp.int32, (NR, BE), 0)
          == src_rel).astype(jnp.bfloat16)
    upd = jnp.dot(oh, msg.astype(jnp.bfloat16),
                  preferred_element_type=jnp.float32)
    acc_ref[pl.ds(n0, NR), :] += upd
    if with_cnt:
        onesm = jnp.where(rows < n_edges,
                          jnp.float32(1.0), jnp.float32(0.0))
        cnt_out[0][pl.ds(n0, NR), :] += jnp.dot(
            oh, jnp.broadcast_to(onesm, (BE, HID)).astype(jnp.bfloat16),
            preferred_element_type=jnp.float32)


def _update_body(h_ref, seg_ref, cnt_ref, gam_ref, bet_ref, o_ref):
    inv = 1.0 / jnp.maximum(cnt_ref[:, 0:1], 1.0)
    hn = h_ref[...] + seg_ref[...] * inv
    mu = jnp.mean(hn, axis=1, keepdims=True)
    var = jnp.mean((hn - mu) ** 2, axis=1, keepdims=True)
    o_ref[...] = ((hn - mu) * lax.rsqrt(var + 1e-5) * gam_ref[...]
                  + bet_ref[...])


def _full(shape):
    return pl.BlockSpec(shape, lambda i: (0, 0))


def kernel(x, params, edge_src, edge_dst):
    n_edges = edge_src.shape[0]
    ep = _round_up(n_edges, SC_WORKERS * SC_CHUNK)
    spread = (jnp.arange(ep - n_edges, dtype=jnp.int32) * 7919) % N_NODES
    dstp = jnp.concatenate([edge_dst.astype(jnp.int32), spread])
    srcp = jnp.concatenate([edge_src.astype(jnp.int32), spread])

    n0s = (srcp[::BE] // 8) * 8
    srcb = srcp.reshape(-1, 1, BE)

    freqs = 1.0 / (10000.0 ** (np.arange(POS_CH, dtype=np.float32) / POS_CH))
    fr_half = np.concatenate([freqs, freqs])
    ph_half = np.concatenate([np.zeros(POS_CH, np.float32),
                              np.full(POS_CH, np.pi / 2, np.float32)])
    fr96 = jnp.asarray(np.tile(fr_half, 3)[None, :])
    ph96 = jnp.asarray(np.tile(ph_half, 3)[None, :])

    def row(v):
        return v.reshape(1, -1)

    h, pe = pl.pallas_call(
        _enc_body,
        grid=(N_NODES // BN,),
        in_specs=[
            pl.BlockSpec((BN, HID), lambda i: (i, 0)),
            _full((HID, HID)), _full((1, HID)),
            _full((HID, HID)), _full((1, HID)),
            _full((1, EMB)), _full((1, EMB)),
        ],
        out_specs=[pl.BlockSpec((BN, HID), lambda i: (i, 0)),
                   pl.BlockSpec((BN, HID), lambda i: (i, 0))],
        out_shape=[jax.ShapeDtypeStruct((N_NODES, HID), jnp.float32),
                   jax.ShapeDtypeStruct((N_NODES, HID), jnp.float32)],
    )(x, params['enc_w1'], row(params['enc_b1']),
      params['enc_w2'], row(params['enc_b2']), fr96, ph96)

    peg = _sc_gather(pe, jnp.concatenate([dstp, srcp]))

    ne_blocks = ep // BE
    n_pad = _round_up(N_NODES + NR, 8)
    cnt = None
    for l in range(NUM_LAYERS):
        hd = _sc_gather(h, dstp)

        w0 = params[f'k{l}_w0']
        w0d = jnp.pad(w0[:EMB], ((0, HID - EMB), (0, 0))).astype(jnp.bfloat16)
        w0s = jnp.pad(w0[EMB:], ((0, HID - EMB), (0, 0))).astype(jnp.bfloat16)
        w1b = params[f'k{l}_w1'].astype(jnp.bfloat16)
        w2b = params[f'k{l}_w2'].astype(jnp.bfloat16)

        with_cnt = l == 0
        out_specs = [pl.BlockSpec((n_pad, HID), lambda i: (0, 0))]
        out_shape = [jax.ShapeDtypeStruct((n_pad, HID), jnp.float32)]
        if with_cnt:
            out_specs.append(pl.BlockSpec((n_pad, HID), lambda i: (0, 0)))
            out_shape.append(jax.ShapeDtypeStruct((n_pad, HID), jnp.float32))
        res = pl.pallas_call(
            functools.partial(_edge_body, n_edges, with_cnt),
            grid=(ne_blocks,),
            in_specs=[
                pl.BlockSpec(memory_space=pltpu.SMEM),
                pl.BlockSpec((BE, HID), lambda i: (i, 0)),
                pl.BlockSpec((BE, HID), lambda i: (i + ne_blocks, 0)),
                pl.BlockSpec((BE, HID), lambda i: (i, 0)),
                pl.BlockSpec((1, 1, BE), lambda i: (i, 0, 0)),
                _full((HID, HID)), _full((HID, HID)), _full((1, HID)),
                _full((HID, 2 * HID)), _full((1, 2 * HID)),
                _full((2 * HID, HID)), _full((1, HID)),
            ],
            out_specs=out_specs,
            out_shape=out_shape,
        )(n0s, peg, peg, hd, srcb,
          w0d, w0s, row(params[f'k{l}_b0']),
          w1b, row(params[f'k{l}_b1']),
          w2b, row(params[f'k{l}_b2']))
        if with_cnt:
            seg, cnt = res
        else:
            seg, = res

        h = pl.pallas_call(
            _update_body,
            grid=(N_NODES // BN,),
            in_specs=[
                pl.BlockSpec((BN, HID), lambda i: (i, 0)),
                pl.BlockSpec((BN, HID), lambda i: (i, 0)),
                pl.BlockSpec((BN, HID), lambda i: (i, 0)),
                _full((1, HID)), _full((1, HID)),
            ],
            out_specs=pl.BlockSpec((BN, HID), lambda i: (i, 0)),
            out_shape=jax.ShapeDtypeStruct((N_NODES, HID), jnp.float32),
        )(h, seg, cnt, row(params[f'ln{l}_g']), row(params[f'ln{l}_b']))

    hw2 = jnp.pad(params['head_w2'], ((0, 0), (0, 5)))
    hb2 = jnp.pad(params['head_b2'], (0, 5))
    out = pl.pallas_call(
        _mlp2_body,
        grid=(N_NODES // BN,),
        in_specs=[
            pl.BlockSpec((BN, HID), lambda i: (i, 0)),
            _full((HID, HID)), _full((1, HID)),
            _full((HID, 8)), _full((1, 8)),
        ],
        out_specs=pl.BlockSpec((BN, 8), lambda i: (i, 0)),
        out_shape=jax.ShapeDtypeStruct((N_NODES, 8), jnp.float32),
    )(h, params['head_w1'], row(params['head_b1']), hw2, row(hb2))

    return out[:, :3]


# speedup vs baseline: 1.4411x; 1.0231x over previous
import functools

import numpy as np
import jax
import jax.numpy as jnp
from jax import lax
from jax.experimental import pallas as pl
from jax.experimental.pallas import tpu as pltpu
from jax.experimental.pallas import tpu_sc as plsc

N_NODES = 10000
POS_CH = 16
EMB = 2 * POS_CH * 3
KIN = 2 * EMB
HID = 128
NUM_LAYERS = 4
SC_WORKERS = 32
SC_CHUNK = 128
BE = 512
BN = 1000


def _round_up(v, m):
    return (v + m - 1) // m * m


def _gelu(v):
    return 0.5 * v * (1.0 + lax.erf(v * np.float32(1.0 / np.sqrt(2.0))))


def _sc_gather(table, idx):
    b_total = idx.shape[0]
    d = table.shape[1]
    b_per_w = b_total // SC_WORKERS
    chunks = b_per_w // SC_CHUNK
    mesh = plsc.VectorSubcoreMesh(core_axis_name="c", subcore_axis_name="s")

    nbuf = min(4, chunks)

    @functools.partial(
        pl.kernel,
        out_type=jax.ShapeDtypeStruct((b_total, d), table.dtype),
        mesh=mesh,
        scratch_types=(
            [pltpu.VMEM((b_per_w,), jnp.int32)]
            + [pltpu.VMEM((SC_CHUNK, d), table.dtype)] * nbuf
            + [pltpu.SemaphoreType.DMA] * (2 * nbuf)
        ),
    )
    def gather_kernel(table_hbm, idx_hbm, out_hbm, idx_v, *rest):
        bufs = rest[:nbuf]
        gsem = rest[nbuf:2 * nbuf]
        wsem = rest[2 * nbuf:]
        wid = lax.axis_index("s") * 2 + lax.axis_index("c")
        base = wid * b_per_w
        pltpu.sync_copy(idx_hbm.at[pl.ds(base, b_per_w)], idx_v)

        def gather_chunk(j):
            return pltpu.async_copy(
                table_hbm.at[idx_v.at[pl.ds(j * SC_CHUNK, SC_CHUNK)]],
                bufs[j % nbuf], gsem[j % nbuf])

        cps = [None] * nbuf
        wbs = [None] * nbuf
        for j in range(nbuf - 1):
            cps[j % nbuf] = gather_chunk(j)
        for j in range(chunks):
            cps[j % nbuf].wait()
            wbs[j % nbuf] = pltpu.async_copy(
                bufs[j % nbuf],
                out_hbm.at[pl.ds(base + j * SC_CHUNK, SC_CHUNK)],
                wsem[j % nbuf])
            nj = j + nbuf - 1
            if nj < chunks:
                if wbs[nj % nbuf] is not None:
                    wbs[nj % nbuf].wait()
                    wbs[nj % nbuf] = None
                cps[nj % nbuf] = gather_chunk(nj)
        for wb in wbs:
            if wb is not None:
                wb.wait()

    return gather_kernel(table, idx)


def _mlp2_body(x_ref, w1_ref, b1_ref, w2_ref, b2_ref, o_ref):
    t = _gelu(jnp.dot(x_ref[...], w1_ref[...],
                      preferred_element_type=jnp.float32) + b1_ref[...])
    o_ref[...] = jnp.dot(t, w2_ref[...],
                         preferred_element_type=jnp.float32) + b2_ref[...]


def _pe_body(x_ref, fr_ref, ph_ref, pe_ref):
    x = x_ref[...]
    cols = [jnp.broadcast_to(x[:, c:c + 1], (x.shape[0], 32))
            for c in range(3)]
    pos_big = jnp.concatenate(cols, axis=1)
    g = jnp.sin(pos_big * fr_ref[...] + ph_ref[...])
    pe_ref[...] = jnp.pad(g, ((0, 0), (0, 32)))


NR = BE + 8


def _edge_body(n_edges, with_cnt, n0s_ref, ped_ref, pes_ref, hd_ref, src_ref,
               w0d_ref, w0s_ref, b0_ref, w1_ref, b1_ref, w2_ref, b2_ref,
               acc_ref, *cnt_out):
    i = pl.program_id(0)

    @pl.when(i == 0)
    def _():
        acc_ref[...] = jnp.zeros_like(acc_ref)
        if with_cnt:
            cnt_out[0][...] = jnp.zeros_like(cnt_out[0])

    t = (jnp.dot(ped_ref[...].astype(jnp.bfloat16), w0d_ref[...],
                 preferred_element_type=jnp.float32)
         + jnp.dot(pes_ref[...].astype(jnp.bfloat16), w0s_ref[...],
                   preferred_element_type=jnp.float32)
         + b0_ref[...])
    t = _gelu(t).astype(jnp.bfloat16)
    t = _gelu(jnp.dot(t, w1_ref[...],
                      preferred_element_type=jnp.float32) + b1_ref[...])
    k = jnp.dot(t.astype(jnp.bfloat16), w2_ref[...],
                preferred_element_type=jnp.float32) + b2_ref[...]

    msg = k * hd_ref[...]
    rows = i * BE + lax.broadcasted_iota(jnp.int32, (BE, 1), 0)
    msg = jnp.where(rows < n_edges, msg, 0.0)

    n0 = n0s_ref[i]
    src_rel = src_ref[0, 0, :].reshape(1, BE) - n0
    oh = (lax.broadcasted_iota(jnp.int32, (NR, BE), 0)
          == src_rel).astype(jnp.bfloat16)
    upd = jnp.dot(oh, msg.astype(jnp.bfloat16),
                  preferred_element_type=jnp.float32)
    acc_ref[pl.ds(n0, NR), :] += upd
    if with_cnt:
        onesm = jnp.where(rows < n_edges,
                          jnp.float32(1.0), jnp.float32(0.0))
        cnt_out[0][pl.ds(n0, NR), :] += jnp.dot(
            oh, jnp.broadcast_to(onesm, (BE, HID)).astype(jnp.bfloat16),
            preferred_element_type=jnp.float32)


def _update_body(h_ref, seg_ref, cnt_ref, gam_ref, bet_ref, o_ref):
    inv = 1.0 / jnp.maximum(cnt_ref[:, 0:1], 1.0)
    hn = h_ref[...] + seg_ref[...] * inv
    mu = jnp.mean(hn, axis=1, keepdims=True)
    var = jnp.mean((hn - mu) ** 2, axis=1, keepdims=True)
    o_ref[...] = ((hn - mu) * lax.rsqrt(var + 1e-5) * gam_ref[...]
                  + bet_ref[...])


def _update_head_body(h_ref, seg_ref, cnt_ref, gam_ref, bet_ref,
                      w1_ref, b1_ref, w2_ref, b2_ref, o_ref):
    inv = 1.0 / jnp.maximum(cnt_ref[:, 0:1], 1.0)
    hn = h_ref[...] + seg_ref[...] * inv
    mu = jnp.mean(hn, axis=1, keepdims=True)
    var = jnp.mean((hn - mu) ** 2, axis=1, keepdims=True)
    y = (hn - mu) * lax.rsqrt(var + 1e-5) * gam_ref[...] + bet_ref[...]
    t = _gelu(jnp.dot(y, w1_ref[...],
                      preferred_element_type=jnp.float32) + b1_ref[...])
    o_ref[...] = jnp.dot(t, w2_ref[...],
                         preferred_element_type=jnp.float32) + b2_ref[...]


def _full(shape):
    return pl.BlockSpec(shape, lambda i: (0, 0))


def kernel(x, params, edge_src, edge_dst):
    n_edges = edge_src.shape[0]
    ep = _round_up(n_edges, SC_WORKERS * SC_CHUNK)
    spread = (jnp.arange(ep - n_edges, dtype=jnp.int32) * 7919) % N_NODES
    dstp = jnp.concatenate([edge_dst.astype(jnp.int32), spread])
    srcp = jnp.concatenate([edge_src.astype(jnp.int32), spread])

    n0s = (srcp[::BE] // 8) * 8
    srcb = srcp.reshape(-1, 1, BE)

    freqs = 1.0 / (10000.0 ** (np.arange(POS_CH, dtype=np.float32) / POS_CH))
    fr_half = np.concatenate([freqs, freqs])
    ph_half = np.concatenate([np.zeros(POS_CH, np.float32),
                              np.full(POS_CH, np.pi / 2, np.float32)])
    fr96 = jnp.asarray(np.tile(fr_half, 3)[None, :])
    ph96 = jnp.asarray(np.tile(ph_half, 3)[None, :])

    def row(v):
        return v.reshape(1, -1)

    pe = pl.pallas_call(
        _pe_body,
        grid=(N_NODES // BN,),
        in_specs=[
            pl.BlockSpec((BN, HID), lambda i: (i, 0)),
            _full((1, EMB)), _full((1, EMB)),
        ],
        out_specs=pl.BlockSpec((BN, HID), lambda i: (i, 0)),
        out_shape=jax.ShapeDtypeStruct((N_NODES, HID), jnp.float32),
    )(x, fr96, ph96)

    peg = _sc_gather(pe, jnp.concatenate([dstp, srcp]))

    h = pl.pallas_call(
        _mlp2_body,
        grid=(N_NODES // BN,),
        in_specs=[
            pl.BlockSpec((BN, HID), lambda i: (i, 0)),
            _full((HID, HID)), _full((1, HID)),
            _full((HID, HID)), _full((1, HID)),
        ],
        out_specs=pl.BlockSpec((BN, HID), lambda i: (i, 0)),
        out_shape=jax.ShapeDtypeStruct((N_NODES, HID), jnp.float32),
    )(x, params['enc_w1'], row(params['enc_b1']),
      params['enc_w2'], row(params['enc_b2']))

    ne_blocks = ep // BE
    n_pad = _round_up(N_NODES + NR, 8)
    cnt = None
    for l in range(NUM_LAYERS):
        hd = _sc_gather(h, dstp)

        w0 = params[f'k{l}_w0']
        w0d = jnp.pad(w0[:EMB], ((0, HID - EMB), (0, 0))).astype(jnp.bfloat16)
        w0s = jnp.pad(w0[EMB:], ((0, HID - EMB), (0, 0))).astype(jnp.bfloat16)
        w1b = params[f'k{l}_w1'].astype(jnp.bfloat16)
        w2b = params[f'k{l}_w2'].astype(jnp.bfloat16)

        with_cnt = l == 0
        out_specs = [pl.BlockSpec((n_pad, HID), lambda i: (0, 0))]
        out_shape = [jax.ShapeDtypeStruct((n_pad, HID), jnp.float32)]
        if with_cnt:
            out_specs.append(pl.BlockSpec((n_pad, HID), lambda i: (0, 0)))
            out_shape.append(jax.ShapeDtypeStruct((n_pad, HID), jnp.float32))
        res = pl.pallas_call(
            functools.partial(_edge_body, n_edges, with_cnt),
            grid=(ne_blocks,),
            in_specs=[
                pl.BlockSpec(memory_space=pltpu.SMEM),
                pl.BlockSpec((BE, HID), lambda i: (i, 0)),
                pl.BlockSpec((BE, HID), lambda i: (i + ne_blocks, 0)),
                pl.BlockSpec((BE, HID), lambda i: (i, 0)),
                pl.BlockSpec((1, 1, BE), lambda i: (i, 0, 0)),
                _full((HID, HID)), _full((HID, HID)), _full((1, HID)),
                _full((HID, 2 * HID)), _full((1, 2 * HID)),
                _full((2 * HID, HID)), _full((1, HID)),
            ],
            out_specs=out_specs,
            out_shape=out_shape,
        )(n0s, peg, peg, hd, srcb,
          w0d, w0s, row(params[f'k{l}_b0']),
          w1b, row(params[f'k{l}_b1']),
          w2b, row(params[f'k{l}_b2']))
        if with_cnt:
            seg, cnt = res
        else:
            seg, = res

        upd_in_specs = [
            pl.BlockSpec((BN, HID), lambda i: (i, 0)),
            pl.BlockSpec((BN, HID), lambda i: (i, 0)),
            pl.BlockSpec((BN, HID), lambda i: (i, 0)),
            _full((1, HID)), _full((1, HID)),
        ]
        if l < NUM_LAYERS - 1:
            h = pl.pallas_call(
                _update_body,
                grid=(N_NODES // BN,),
                in_specs=upd_in_specs,
                out_specs=pl.BlockSpec((BN, HID), lambda i: (i, 0)),
                out_shape=jax.ShapeDtypeStruct((N_NODES, HID), jnp.float32),
            )(h, seg, cnt, row(params[f'ln{l}_g']), row(params[f'ln{l}_b']))
        else:
            hw2 = jnp.pad(params['head_w2'], ((0, 0), (0, 5)))
            hb2 = jnp.pad(params['head_b2'], (0, 5))
            out = pl.pallas_call(
                _update_head_body,
                grid=(N_NODES // BN,),
                in_specs=upd_in_specs + [
                    _full((HID, HID)), _full((1, HID)),
                    _full((HID, 8)), _full((1, 8)),
                ],
                out_specs=pl.BlockSpec((BN, 8), lambda i: (i, 0)),
                out_shape=jax.ShapeDtypeStruct((N_NODES, 8), jnp.float32),
            )(h, seg, cnt, row(params[f'ln{l}_g']), row(params[f'ln{l}_b']),
              params['head_w1'], row(params['head_b1']), hw2, row(hb2))

    return out[:, :3]
